# baseline (speedup 1.0000x reference)
import functools

import jax
import jax.numpy as jnp
from jax import lax
from jax.experimental import pallas as pl
from jax.experimental.pallas import tpu as pltpu
from jax.experimental.pallas import tpu_sc as plsc

N = 50000
E = 800000
DD = 1536
HID = 200
F1 = 512
F2 = 128
NCORE = 2
NSUB = 16
LANES = 16

EPT = 51200
SB = 2048
NITER = SB // LANES
NSB = EPT // SB
E_PAD = EPT * NSUB
GB = 128
NBMAX = SB // GB

CHUNK1 = 2816
NCHUNK1 = 18
NPAD1 = CHUNK1 * NCHUNK1
CHUNK2 = 8448
NCHUNK2 = 6
NPAD2 = CHUNK2 * NCHUNK2

BM = 512


def _make_seg_body(CHUNK, NCHUNK, WS):
    rows_t = CHUNK // NSUB
    GBK = 128 // WS
    gsh = GBK.bit_length() - 1

    def body(feat, srce, dste, zeros, out, srcs, dsts, srcm, dstm,
             srcx0, dstx0, rows0, acc, semg0, sems0):
        srcx = (srcx0, srcx0)
        dstx = (dstx0, dstx0)
        rows = (rows0, rows0)
        cid = lax.axis_index("c")
        sid = lax.axis_index("s")

        edge_base = sid * EPT
        r0 = sid * rows_t
        iota = lax.iota(jnp.int32, LANES)
        zpad = jnp.zeros((LANES,), jnp.int32)
        tspread = 128 // WS
        qq = iota // WS if WS > 1 else iota
        rr = iota - qq * WS
        semg = (semg0, semg0)
        sems = (sems0, sems0)

        def expand(b, p):
            if WS > 1:
                bsp = jnp.zeros((LANES,), jnp.int32) + b
                for k in range(128 // LANES):
                    ki = k * (LANES // WS) + qq
                    sv = plsc.load_gather(srcm, [bsp, ki])
                    dv = plsc.load_gather(dstm, [bsp, ki])
                    srcx[p][0, pl.ds(k * LANES, LANES)] = sv * WS + rr
                    dstx[p][0, pl.ds(k * LANES, LANES)] = dv * WS + rr

        def sidx(b, p):
            return srcx[p].at[0] if WS > 1 else srcm.at[b]

        def didx(b, p):
            return dstx[p].at[0] if WS > 1 else dstm.at[b]

        def g_start(b, p):
            expand(b, p)
            return pltpu.async_copy(feat.at[sidx(b, p)], rows[p], semg[p])

        def s_start(b, p):
            return pltpu.async_copy(rows[p], acc.at[didx(b, p)],
                                    sems[p], add=True)

        def chunk_body(ci, carry):
            base = (ci * NCORE + cid) * CHUNK
            pltpu.sync_copy(zeros, acc.at[pl.ds(r0 * WS, rows_t * WS)])
            plsc.subcore_barrier()

            def sb_body(sb, c2):
                eb = edge_base + sb * SB
                pltpu.sync_copy(srce.at[pl.ds(eb, SB)], srcs)
                pltpu.sync_copy(dste.at[pl.ds(eb, SB)], dsts)

                def filt(i, cnt):
                    d = dsts[pl.ds(i * LANES, LANES)]
                    s = srcs[pl.ds(i * LANES, LANES)]
                    dl = d - base
                    m = (dl >= 0) & (dl < CHUNK)
                    pos = cnt + plsc.cumsum(m.astype(jnp.int32)) - 1
                    hi = pos >> gsh
                    lo = pos & (GBK - 1)
                    plsc.store_scatter(srcm, [hi, lo], s, mask=m)
                    plsc.store_scatter(dstm, [hi, lo], dl, mask=m)
                    return pos[LANES - 1] + 1

                cnt = lax.fori_loop(0, NITER, filt, 0)
                for j in range(2 * (GBK // LANES) if GBK >= LANES else 2):
                    pos = cnt + j * LANES + iota
                    hi = pos >> gsh
                    lo = pos & (GBK - 1)
                    tpad = CHUNK + (pos & (tspread - 1))
                    plsc.store_scatter(srcm, [hi, lo], zpad)
                    plsc.store_scatter(dstm, [hi, lo], tpad)
                nb = ((cnt + 2 * GBK - 1) // (2 * GBK)) * 2

                def batch(b, c3):
                    d0 = g_start(b, 0)
                    d0.wait()
                    pltpu.sync_copy(rows[0], acc.at[didx(b, 0)],
                                    add=True)
                    return c3
                lax.fori_loop(0, nb, batch, 0)
                return c2
            lax.fori_loop(0, NSB, sb_body, 0)
            plsc.subcore_barrier()
            pltpu.sync_copy(acc.at[pl.ds(r0 * WS, rows_t * WS)],
                            out.at[pl.ds((base + r0) * WS, rows_t * WS)])
            return carry
        lax.fori_loop(0, NCHUNK // NCORE, chunk_body, 0)

    return body


def _seg(feat, src_p, dst_p, CHUNK, NCHUNK, WS):
    rows_t = CHUNK // NSUB
    GBK = 128 // WS
    nba = SB // GBK + 2
    k = functools.partial(
        pl.kernel,
        out_type=jax.ShapeDtypeStruct((CHUNK * NCHUNK * WS, 128),
                                      jnp.float32),
        mesh=plsc.VectorSubcoreMesh(core_axis_name="c", subcore_axis_name="s",
                                    num_cores=NCORE, num_subcores=NSUB),
        compiler_params=pltpu.CompilerParams(needs_layout_passes=False),
        scratch_types=[
            pltpu.VMEM((SB,), jnp.int32),
            pltpu.VMEM((SB,), jnp.int32),
            pltpu.VMEM((nba, GBK), jnp.int32),
            pltpu.VMEM((nba, GBK), jnp.int32),
            pltpu.VMEM((1, 128), jnp.int32),
            pltpu.VMEM((1, 128), jnp.int32),
            pltpu.VMEM((128, 128), jnp.float32),
            pltpu.VMEM_SHARED(((CHUNK + 128 // WS) * WS, 128), jnp.float32),
            pltpu.SemaphoreType.DMA,
            pltpu.SemaphoreType.DMA,
        ],
    )(_make_seg_body(CHUNK, NCHUNK, WS))
    zeros = jnp.zeros((rows_t * WS, 128), jnp.float32)
    return k(feat, src_p, dst_p, zeros)


def _seg1(feat, src_p, dst_p):
    return _seg(feat, src_p, dst_p, CHUNK1, NCHUNK1, 4)


def _seg2(feat, src_p, dst_p):
    return _seg(feat, src_p, dst_p, CHUNK2, NCHUNK2, 1)


def _mm_body(xb, wb, ob):
    y = jnp.dot(xb[...], wb[...], preferred_element_type=jnp.float32)
    ob[...] = y.reshape(ob.shape)


def _l2_body(ab, b1b, wb, ob):
    a = ab[...].reshape(BM, F1)
    h = jnp.maximum(a + b1b[...], 0.0)
    ob[...] = jnp.dot(h, wb[...], preferred_element_type=jnp.float32)


def _vote_body(pb, b2b, ob):
    s = pb[...] + b2b[...]
    a, b = s[:, 0:1], s[:, 1:2]
    c, d = s[:, 2:3], s[:, 3:4]
    p0 = jax.nn.sigmoid(a - b) + jax.nn.sigmoid(c - d)
    p1 = jax.nn.sigmoid(b - a) + jax.nn.sigmoid(d - c)
    ob[...] = 0.5 * jnp.concatenate([p0, p1], axis=1)


def _cdiv(a, b):
    return (a + b - 1) // b


def kernel(x, edge_index, W1a, b1a, W2a, b2a, W1b, b1b, W2b, b2b):
    f32 = jnp.float32
    src = edge_index[0]
    dst = edge_index[1]

    Wcat = jnp.zeros((DD, F1), f32).at[:768, :HID].set(W1a)
    Wcat = Wcat.at[768:, HID:2 * HID].set(W1b)
    b1cat = jnp.zeros((1, F1), f32).at[0, :HID].set(b1a)
    b1cat = b1cat.at[0, HID:2 * HID].set(b1b)
    W2cat = jnp.zeros((F1, F2), f32).at[:HID, 0:2].set(W2a)
    W2cat = W2cat.at[HID:2 * HID, 2:4].set(W2b)
    b2cat = jnp.zeros((1, F2), f32).at[0, 0:2].set(b2a).at[0, 2:4].set(b2b)

    pad = E_PAD - E
    src_p = jnp.concatenate([src, jnp.zeros((pad,), jnp.int32)])
    dst_p = jnp.concatenate([dst, jnp.full((pad,), 1 << 20, jnp.int32)])

    proj = pl.pallas_call(
        _mm_body,
        grid=(_cdiv(N, BM),),
        in_specs=[pl.BlockSpec((BM, DD), lambda i: (i, 0)),
                  pl.BlockSpec((DD, F1), lambda i: (0, 0))],
        out_specs=pl.BlockSpec((4 * BM, 128), lambda i: (i, 0)),
        out_shape=jax.ShapeDtypeStruct((4 * _cdiv(N, BM) * BM, 128), f32),
    )(x, Wcat)

    agg1 = _seg1(proj, src_p, dst_p)

    z = pl.pallas_call(
        _l2_body,
        grid=(NPAD1 // BM,),
        in_specs=[pl.BlockSpec((4 * BM, 128), lambda i: (i, 0)),
                  pl.BlockSpec((1, F1), lambda i: (0, 0)),
                  pl.BlockSpec((F1, F2), lambda i: (0, 0))],
        out_specs=pl.BlockSpec((BM, F2), lambda i: (i, 0)),
        out_shape=jax.ShapeDtypeStruct((NPAD1, F2), f32),
    )(agg1, b1cat, W2cat)

    agg2 = _seg2(z, src_p, dst_p)

    out = pl.pallas_call(
        _vote_body,
        grid=(_cdiv(N, BM),),
        in_specs=[pl.BlockSpec((BM, F2), lambda i: (i, 0)),
                  pl.BlockSpec((1, F2), lambda i: (0, 0))],
        out_specs=pl.BlockSpec((BM, 2), lambda i: (i, 0)),
        out_shape=jax.ShapeDtypeStruct((N, 2), f32),
    )(agg2, b2cat)

    return out


# speedup vs baseline: 5.5116x; 5.5116x over previous
import functools

import jax
import jax.numpy as jnp
from jax import lax
from jax.experimental import pallas as pl
from jax.experimental.pallas import tpu as pltpu
from jax.experimental.pallas import tpu_sc as plsc

N = 50000
E = 800000
DD = 1536
HID = 200
F1 = 512
F2 = 128
NCORE = 2
NSUB = 16
LANES = 16

EPT = 51200
SB = 2048
NITER = SB // LANES
NSB = EPT // SB
E_PAD = EPT * NSUB
GB = 128
NBMAX = SB // GB

CHUNK1 = 2816
NCHUNK1 = 18
NPAD1 = CHUNK1 * NCHUNK1
CHUNK2 = 8448
NCHUNK2 = 6
NPAD2 = CHUNK2 * NCHUNK2

BM = 512


def _make_seg_body(CHUNK, NCHUNK, WS):
    rows_t = CHUNK // NSUB
    GBK = 128 // WS
    gsh = GBK.bit_length() - 1

    def body(feat, srce, dste, zeros, out, srcs, dsts, srcm, dstm,
             srcx0, dstx0, rows0, acc, semg0, sems0):
        srcx = (srcx0, srcx0)
        dstx = (dstx0, dstx0)
        rows = (rows0, rows0)
        cid = lax.axis_index("c")
        sid = lax.axis_index("s")

        edge_base = sid * EPT
        r0 = sid * rows_t
        iota = lax.iota(jnp.int32, LANES)
        zpad = jnp.zeros((LANES,), jnp.int32)
        tspread = 128 // WS
        qq = iota // WS if WS > 1 else iota
        rr = iota - qq * WS
        semg = (semg0, semg0)
        sems = (sems0, sems0)

        def expand(b, p):
            if WS > 1:
                bsp = jnp.zeros((LANES,), jnp.int32) + b
                for k in range(128 // LANES):
                    ki = k * (LANES // WS) + qq
                    sv = plsc.load_gather(srcm, [bsp, ki])
                    dv = plsc.load_gather(dstm, [bsp, ki])
                    srcx[p][0, pl.ds(k * LANES, LANES)] = sv * WS + rr
                    dstx[p][0, pl.ds(k * LANES, LANES)] = dv * WS + rr

        def sidx(b, p):
            return srcx[p].at[0] if WS > 1 else srcm.at[b]

        def didx(b, p):
            return dstx[p].at[0] if WS > 1 else dstm.at[b]

        def g_start(b, p):
            expand(b, p)
            return pltpu.async_copy(feat.at[sidx(b, p)], rows[p], semg[p])

        def s_start(b, p):
            return pltpu.async_copy(rows[p], acc.at[didx(b, p)],
                                    sems[p], add=True)

        def chunk_body(ci, carry):
            base = (ci * NCORE + cid) * CHUNK
            pltpu.sync_copy(zeros, acc.at[pl.ds(r0 * WS, rows_t * WS)])
            plsc.subcore_barrier()

            def sb_body(sb, c2):
                eb = edge_base + sb * SB
                pltpu.sync_copy(srce.at[pl.ds(eb, SB)], srcs)
                pltpu.sync_copy(dste.at[pl.ds(eb, SB)], dsts)

                def filt(i, cnt):
                    d = dsts[pl.ds(i * LANES, LANES)]
                    s = srcs[pl.ds(i * LANES, LANES)]
                    dl = d - base
                    m = (dl >= 0) & (dl < CHUNK)
                    pos = cnt + plsc.cumsum(m.astype(jnp.int32)) - 1
                    hi = pos >> gsh
                    lo = pos & (GBK - 1)
                    plsc.store_scatter(srcm, [hi, lo], s, mask=m)
                    plsc.store_scatter(dstm, [hi, lo], dl, mask=m)
                    return pos[LANES - 1] + 1

                cnt = lax.fori_loop(0, NITER, filt, 0)
                for j in range(2 * (GBK // LANES) if GBK >= LANES else 2):
                    pos = cnt + j * LANES + iota
                    hi = pos >> gsh
                    lo = pos & (GBK - 1)
                    tpad = CHUNK + (pos & (tspread - 1))
                    plsc.store_scatter(srcm, [hi, lo], pos & (tspread - 1))
                    plsc.store_scatter(dstm, [hi, lo], tpad)
                nb = ((cnt + 2 * GBK - 1) // (2 * GBK)) * 2

                def batch(b, c3):
                    d0 = g_start(b, 0)
                    d0.wait()
                    pltpu.sync_copy(rows[0], acc.at[didx(b, 0)],
                                    add=True)
                    return c3
                lax.fori_loop(0, nb, batch, 0)
                return c2
            lax.fori_loop(0, NSB, sb_body, 0)
            plsc.subcore_barrier()
            pltpu.sync_copy(acc.at[pl.ds(r0 * WS, rows_t * WS)],
                            out.at[pl.ds((base + r0) * WS, rows_t * WS)])
            return carry
        lax.fori_loop(0, NCHUNK // NCORE, chunk_body, 0)

    return body


def _seg(feat, src_p, dst_p, CHUNK, NCHUNK, WS):
    rows_t = CHUNK // NSUB
    GBK = 128 // WS
    nba = SB // GBK + 2
    k = functools.partial(
        pl.kernel,
        out_type=jax.ShapeDtypeStruct((CHUNK * NCHUNK * WS, 128),
                                      jnp.float32),
        mesh=plsc.VectorSubcoreMesh(core_axis_name="c", subcore_axis_name="s",
                                    num_cores=NCORE, num_subcores=NSUB),
        compiler_params=pltpu.CompilerParams(needs_layout_passes=False),
        scratch_types=[
            pltpu.VMEM((SB,), jnp.int32),
            pltpu.VMEM((SB,), jnp.int32),
            pltpu.VMEM((nba, GBK), jnp.int32),
            pltpu.VMEM((nba, GBK), jnp.int32),
            pltpu.VMEM((1, 128), jnp.int32),
            pltpu.VMEM((1, 128), jnp.int32),
            pltpu.VMEM((128, 128), jnp.float32),
            pltpu.VMEM_SHARED(((CHUNK + 128 // WS) * WS, 128), jnp.float32),
            pltpu.SemaphoreType.DMA,
            pltpu.SemaphoreType.DMA,
        ],
    )(_make_seg_body(CHUNK, NCHUNK, WS))
    zeros = jnp.zeros((rows_t * WS, 128), jnp.float32)
    return k(feat, src_p, dst_p, zeros)


def _seg1(feat, src_p, dst_p):
    return _seg(feat, src_p, dst_p, CHUNK1, NCHUNK1, 4)


def _seg2(feat, src_p, dst_p):
    return _seg(feat, src_p, dst_p, CHUNK2, NCHUNK2, 1)


def _mm_body(xb, wb, ob):
    y = jnp.dot(xb[...], wb[...], preferred_element_type=jnp.float32)
    ob[...] = y.reshape(ob.shape)


def _l2_body(ab, b1b, wb, ob):
    a = ab[...].reshape(BM, F1)
    h = jnp.maximum(a + b1b[...], 0.0)
    ob[...] = jnp.dot(h, wb[...], preferred_element_type=jnp.float32)


def _vote_body(pb, b2b, ob):
    s = pb[...] + b2b[...]
    a, b = s[:, 0:1], s[:, 1:2]
    c, d = s[:, 2:3], s[:, 3:4]
    p0 = jax.nn.sigmoid(a - b) + jax.nn.sigmoid(c - d)
    p1 = jax.nn.sigmoid(b - a) + jax.nn.sigmoid(d - c)
    ob[...] = 0.5 * jnp.concatenate([p0, p1], axis=1)


def _cdiv(a, b):
    return (a + b - 1) // b


def kernel(x, edge_index, W1a, b1a, W2a, b2a, W1b, b1b, W2b, b2b):
    f32 = jnp.float32
    src = edge_index[0]
    dst = edge_index[1]

    Wcat = jnp.zeros((DD, F1), f32).at[:768, :HID].set(W1a)
    Wcat = Wcat.at[768:, HID:2 * HID].set(W1b)
    b1cat = jnp.zeros((1, F1), f32).at[0, :HID].set(b1a)
    b1cat = b1cat.at[0, HID:2 * HID].set(b1b)
    W2cat = jnp.zeros((F1, F2), f32).at[:HID, 0:2].set(W2a)
    W2cat = W2cat.at[HID:2 * HID, 2:4].set(W2b)
    b2cat = jnp.zeros((1, F2), f32).at[0, 0:2].set(b2a).at[0, 2:4].set(b2b)

    pad = E_PAD - E
    src_p = jnp.concatenate([src, jnp.zeros((pad,), jnp.int32)])
    dst_p = jnp.concatenate([dst, jnp.full((pad,), 1 << 20, jnp.int32)])

    proj = pl.pallas_call(
        _mm_body,
        grid=(_cdiv(N, BM),),
        in_specs=[pl.BlockSpec((BM, DD), lambda i: (i, 0)),
                  pl.BlockSpec((DD, F1), lambda i: (0, 0))],
        out_specs=pl.BlockSpec((4 * BM, 128), lambda i: (i, 0)),
        out_shape=jax.ShapeDtypeStruct((4 * _cdiv(N, BM) * BM, 128), f32),
    )(x, Wcat)

    agg1 = _seg1(proj, src_p, dst_p)

    z = pl.pallas_call(
        _l2_body,
        grid=(NPAD1 // BM,),
        in_specs=[pl.BlockSpec((4 * BM, 128), lambda i: (i, 0)),
                  pl.BlockSpec((1, F1), lambda i: (0, 0)),
                  pl.BlockSpec((F1, F2), lambda i: (0, 0))],
        out_specs=pl.BlockSpec((BM, F2), lambda i: (i, 0)),
        out_shape=jax.ShapeDtypeStruct((NPAD1, F2), f32),
    )(agg1, b1cat, W2cat)

    agg2 = _seg2(z, src_p, dst_p)

    out = pl.pallas_call(
        _vote_body,
        grid=(_cdiv(N, BM),),
        in_specs=[pl.BlockSpec((BM, F2), lambda i: (i, 0)),
                  pl.BlockSpec((1, F2), lambda i: (0, 0))],
        out_specs=pl.BlockSpec((BM, 2), lambda i: (i, 0)),
        out_shape=jax.ShapeDtypeStruct((N, 2), f32),
    )(agg2, b2cat)

    return out


# speedup vs baseline: 5.6692x; 1.0286x over previous
import functools

import jax
import jax.numpy as jnp
from jax import lax
from jax.experimental import pallas as pl
from jax.experimental.pallas import tpu as pltpu
from jax.experimental.pallas import tpu_sc as plsc

N = 50000
E = 800000
DD = 1536
HID = 200
F1 = 512
F2 = 128
NCORE = 2
NSUB = 16
LANES = 16

EPT = 51200
SB = 2048
NITER = SB // LANES
NSB = EPT // SB
E_PAD = EPT * NSUB
GB = 128
NBMAX = SB // GB

CHUNK1 = 2304
NCHUNK1 = 22
NPAD1 = CHUNK1 * NCHUNK1
CHUNK2 = 8448
NCHUNK2 = 6
NPAD2 = CHUNK2 * NCHUNK2

BM = 512


def _make_seg_body(CHUNK, NCHUNK, WS):
    rows_t = CHUNK // NSUB
    GBK = 128 // WS
    gsh = GBK.bit_length() - 1

    def body(feat, srce, dste, zeros, out, srcs, dsts, srcm, dstm,
             srcx0, srcx1, dstx0, dstx1, rows0, rows1, acc,
             semg0, semg1, sems0, sems1):
        srcx = (srcx0, srcx1)
        dstx = (dstx0, dstx1)
        rows = (rows0, rows1)
        cid = lax.axis_index("c")
        sid = lax.axis_index("s")

        edge_base = sid * EPT
        r0 = sid * rows_t
        iota = lax.iota(jnp.int32, LANES)
        zpad = jnp.zeros((LANES,), jnp.int32)
        tspread = 128 // WS
        qq = iota // WS if WS > 1 else iota
        rr = iota - qq * WS
        semg = (semg0, semg1)
        sems = (sems0, sems1)

        def expand(b, p):
            if WS > 1:
                bsp = jnp.zeros((LANES,), jnp.int32) + b
                for k in range(128 // LANES):
                    ki = k * (LANES // WS) + qq
                    sv = plsc.load_gather(srcm, [bsp, ki])
                    dv = plsc.load_gather(dstm, [bsp, ki])
                    srcx[p][0, pl.ds(k * LANES, LANES)] = sv * WS + rr
                    dstx[p][0, pl.ds(k * LANES, LANES)] = dv * WS + rr

        def sidx(b, p):
            return srcx[p].at[0] if WS > 1 else srcm.at[b]

        def didx(b, p):
            return dstx[p].at[0] if WS > 1 else dstm.at[b]

        def g_start(b, p):
            expand(b, p)
            return pltpu.async_copy(feat.at[sidx(b, p)], rows[p], semg[p])

        def s_start(b, p):
            return pltpu.async_copy(rows[p], acc.at[didx(b, p)],
                                    sems[p], add=True)

        def chunk_body(ci, carry):
            base = (ci * NCORE + cid) * CHUNK
            pltpu.sync_copy(zeros, acc.at[pl.ds(r0 * WS, rows_t * WS)])
            plsc.subcore_barrier()

            def sb_body(sb, c2):
                eb = edge_base + sb * SB
                pltpu.sync_copy(srce.at[pl.ds(eb, SB)], srcs)
                pltpu.sync_copy(dste.at[pl.ds(eb, SB)], dsts)

                def filt(i, cnt):
                    d = dsts[pl.ds(i * LANES, LANES)]
                    s = srcs[pl.ds(i * LANES, LANES)]
                    dl = d - base
                    m = (dl >= 0) & (dl < CHUNK)
                    pos = cnt + plsc.cumsum(m.astype(jnp.int32)) - 1
                    hi = pos >> gsh
                    lo = pos & (GBK - 1)
                    plsc.store_scatter(srcm, [hi, lo], s, mask=m)
                    plsc.store_scatter(dstm, [hi, lo], dl, mask=m)
                    return pos[LANES - 1] + 1

                cnt = lax.fori_loop(0, NITER, filt, 0)
                for j in range(2 * (GBK // LANES) if GBK >= LANES else 2):
                    pos = cnt + j * LANES + iota
                    hi = pos >> gsh
                    lo = pos & (GBK - 1)
                    tpad = CHUNK + (pos & (tspread - 1))
                    plsc.store_scatter(srcm, [hi, lo], pos & (tspread - 1))
                    plsc.store_scatter(dstm, [hi, lo], tpad)
                nb = ((cnt + 2 * GBK - 1) // (2 * GBK)) * 2

                def pair(g, c4):
                    b0 = 2 * g
                    d0 = g_start(b0, 0)
                    d1 = g_start(b0 + 1, 1)
                    d0.wait()
                    s0 = s_start(b0, 0)
                    d1.wait()
                    s1 = s_start(b0 + 1, 1)
                    s0.wait()
                    s1.wait()
                    return c4
                lax.fori_loop(0, nb >> 1, pair, 0)
                return c2
            lax.fori_loop(0, NSB, sb_body, 0)
            plsc.subcore_barrier()
            pltpu.sync_copy(acc.at[pl.ds(r0 * WS, rows_t * WS)],
                            out.at[pl.ds((base + r0) * WS, rows_t * WS)])
            return carry
        lax.fori_loop(0, NCHUNK // NCORE, chunk_body, 0)

    return body


def _seg(feat, src_p, dst_p, CHUNK, NCHUNK, WS):
    rows_t = CHUNK // NSUB
    GBK = 128 // WS
    nba = SB // GBK + 2
    k = functools.partial(
        pl.kernel,
        out_type=jax.ShapeDtypeStruct((CHUNK * NCHUNK * WS, 128),
                                      jnp.float32),
        mesh=plsc.VectorSubcoreMesh(core_axis_name="c", subcore_axis_name="s",
                                    num_cores=NCORE, num_subcores=NSUB),
        compiler_params=pltpu.CompilerParams(needs_layout_passes=False),
        scratch_types=[
            pltpu.VMEM((SB,), jnp.int32),
            pltpu.VMEM((SB,), jnp.int32),
            pltpu.VMEM((nba, GBK), jnp.int32),
            pltpu.VMEM((nba, GBK), jnp.int32),
            pltpu.VMEM((1, 128), jnp.int32),
            pltpu.VMEM((1, 128), jnp.int32),
            pltpu.VMEM((1, 128), jnp.int32),
            pltpu.VMEM((1, 128), jnp.int32),
            pltpu.VMEM((128, 128), jnp.float32),
            pltpu.VMEM((128, 128), jnp.float32),
            pltpu.VMEM_SHARED(((CHUNK + 128 // WS) * WS, 128), jnp.float32),
            pltpu.SemaphoreType.DMA,
            pltpu.SemaphoreType.DMA,
            pltpu.SemaphoreType.DMA,
            pltpu.SemaphoreType.DMA,
        ],
    )(_make_seg_body(CHUNK, NCHUNK, WS))
    zeros = jnp.zeros((rows_t * WS, 128), jnp.float32)
    return k(feat, src_p, dst_p, zeros)


def _seg1(feat, src_p, dst_p):
    return _seg(feat, src_p, dst_p, CHUNK1, NCHUNK1, 4)


def _seg2(feat, src_p, dst_p):
    return _seg(feat, src_p, dst_p, CHUNK2, NCHUNK2, 1)


def _mm_body(xb, wb, ob):
    y = jnp.dot(xb[...], wb[...], preferred_element_type=jnp.float32)
    ob[...] = y.reshape(ob.shape)


def _l2_body(ab, b1b, wb, ob):
    a = ab[...].reshape(BM, F1)
    h = jnp.maximum(a + b1b[...], 0.0)
    ob[...] = jnp.dot(h, wb[...], preferred_element_type=jnp.float32)


def _vote_body(pb, b2b, ob):
    s = pb[...] + b2b[...]
    a, b = s[:, 0:1], s[:, 1:2]
    c, d = s[:, 2:3], s[:, 3:4]
    p0 = jax.nn.sigmoid(a - b) + jax.nn.sigmoid(c - d)
    p1 = jax.nn.sigmoid(b - a) + jax.nn.sigmoid(d - c)
    ob[...] = 0.5 * jnp.concatenate([p0, p1], axis=1)


def _cdiv(a, b):
    return (a + b - 1) // b


def kernel(x, edge_index, W1a, b1a, W2a, b2a, W1b, b1b, W2b, b2b):
    f32 = jnp.float32
    src = edge_index[0]
    dst = edge_index[1]

    Wcat = jnp.zeros((DD, F1), f32).at[:768, :HID].set(W1a)
    Wcat = Wcat.at[768:, HID:2 * HID].set(W1b)
    b1cat = jnp.zeros((1, F1), f32).at[0, :HID].set(b1a)
    b1cat = b1cat.at[0, HID:2 * HID].set(b1b)
    W2cat = jnp.zeros((F1, F2), f32).at[:HID, 0:2].set(W2a)
    W2cat = W2cat.at[HID:2 * HID, 2:4].set(W2b)
    b2cat = jnp.zeros((1, F2), f32).at[0, 0:2].set(b2a).at[0, 2:4].set(b2b)

    pad = E_PAD - E
    src_p = jnp.concatenate([src, jnp.zeros((pad,), jnp.int32)])
    dst_p = jnp.concatenate([dst, jnp.full((pad,), 1 << 20, jnp.int32)])

    proj = pl.pallas_call(
        _mm_body,
        grid=(_cdiv(N, BM),),
        in_specs=[pl.BlockSpec((BM, DD), lambda i: (i, 0)),
                  pl.BlockSpec((DD, F1), lambda i: (0, 0))],
        out_specs=pl.BlockSpec((4 * BM, 128), lambda i: (i, 0)),
        out_shape=jax.ShapeDtypeStruct((4 * _cdiv(N, BM) * BM, 128), f32),
    )(x, Wcat)

    agg1 = _seg1(proj, src_p, dst_p)

    z = pl.pallas_call(
        _l2_body,
        grid=(NPAD1 // BM,),
        in_specs=[pl.BlockSpec((4 * BM, 128), lambda i: (i, 0)),
                  pl.BlockSpec((1, F1), lambda i: (0, 0)),
                  pl.BlockSpec((F1, F2), lambda i: (0, 0))],
        out_specs=pl.BlockSpec((BM, F2), lambda i: (i, 0)),
        out_shape=jax.ShapeDtypeStruct((NPAD1, F2), f32),
    )(agg1, b1cat, W2cat)

    agg2 = _seg2(z, src_p, dst_p)

    out = pl.pallas_call(
        _vote_body,
        grid=(_cdiv(N, BM),),
        in_specs=[pl.BlockSpec((BM, F2), lambda i: (i, 0)),
                  pl.BlockSpec((1, F2), lambda i: (0, 0))],
        out_specs=pl.BlockSpec((BM, 2), lambda i: (i, 0)),
        out_shape=jax.ShapeDtypeStruct((N, 2), f32),
    )(agg2, b2cat)

    return out
